# baseline (speedup 1.0000x reference)
import jax
import jax.numpy as jnp
from jax import lax
from jax.experimental import pallas as pl
from jax.experimental.pallas import tpu as pltpu
from jax.experimental.pallas import tpu_sc as plsc

N = 10000
E = 320000
D = 128

NC = 2
NS = 16
NW = NC * NS
CH = 128
NCH = 80
EP = NW * NCH * CH
EPW = EP // NW
NB = 2
NOUT = NCH // NB
NP = 10240
RPT = NP // NS
ZDR = 80

_MESH = plsc.VectorSubcoreMesh(core_axis_name="c", subcore_axis_name="s")
_CP = pltpu.CompilerParams(use_tc_tiling_on_sc=False)


def _bcast_lane(vec16, r):
    idx = jnp.full((16, 1), r, dtype=jnp.int32)
    dn = lax.GatherDimensionNumbers(
        offset_dims=(), collapsed_slice_dims=(0,), start_index_map=(0,))
    return lax.gather(vec16, idx, dn, (1,),
                      mode=lax.GatherScatterMode.PROMISE_IN_BOUNDS)


def _deg_body(dst_hbm, out_deg, dst_all, ones_v, deg_sh, dsem):
    cid = lax.axis_index("c")
    sid = lax.axis_index("s")
    wid = sid * NC + cid
    rbase = sid * RPT

    def zdrow(r, _):
        ones_v[r, pl.ds(0, 16)] = jnp.zeros((16,), jnp.float32)
        return 0
    lax.fori_loop(0, CH, zdrow, 0)
    for k in range(RPT // ZDR):
        pltpu.sync_copy(ones_v.at[pl.ds(0, ZDR), :],
                        deg_sh.at[pl.ds(rbase + k * ZDR, ZDR), :])

    def orow(r, _):
        ones_v[r, pl.ds(0, 16)] = jnp.ones((16,), jnp.float32)
        return 0
    lax.fori_loop(0, CH, orow, 0)
    pltpu.sync_copy(dst_hbm.at[wid], dst_all)
    plsc.subcore_barrier()

    def chunk(i, _):
        pltpu.async_copy(ones_v, deg_sh.at[dst_all.at[i]], dsem, add=True)
        return 0
    lax.fori_loop(0, NCH, chunk, 0)

    def drain(i, _):
        pltpu.make_async_copy(ones_v, deg_sh.at[dst_all.at[0]], dsem).wait()
        return 0
    lax.fori_loop(0, NCH, drain, 0)

    plsc.subcore_barrier()
    pltpu.sync_copy(deg_sh.at[pl.ds(rbase, RPT), :],
                    out_deg.at[cid, pl.ds(rbase, RPT), :])


_deg = pl.kernel(
    _deg_body,
    out_type=jax.ShapeDtypeStruct((NC, NP, 16), jnp.float32),
    mesh=_MESH,
    scratch_types=[
        pltpu.VMEM((NCH, CH), jnp.int32),
        pltpu.VMEM((CH, 16), jnp.float32),
        pltpu.VMEM_SHARED((NP, 16), jnp.float32),
        pltpu.SemaphoreType.DMA,
    ],
    compiler_params=_CP,
)


def _agg_body(x_hbm, src_hbm, dst_hbm, ew_hbm, out_sum, esrc, edst, eew,
              acc_sh, *tail):
    rows = list(tail[:NB])
    gsems = list(tail[NB:2 * NB])
    ssems = list(tail[2 * NB:3 * NB])
    esem = tail[3 * NB]
    cid = lax.axis_index("c")
    sid = lax.axis_index("s")
    wid = sid * NC + cid
    rbase = sid * RPT

    zbuf = rows[0]

    def zrow(r, _):
        for c8 in range(D // 16):
            zbuf[r, pl.ds(c8 * 16, 16)] = jnp.zeros((16,), jnp.float32)
        return 0
    lax.fori_loop(0, CH, zrow, 0)
    for k in range(RPT // CH):
        pltpu.sync_copy(zbuf, acc_sh.at[pl.ds(rbase + k * CH, CH), :])

    pltpu.sync_copy(src_hbm.at[wid, pl.ds(0, NB)], esrc.at[0])
    pltpu.sync_copy(dst_hbm.at[wid, pl.ds(0, NB)], edst.at[0])
    pltpu.sync_copy(ew_hbm.at[wid, pl.ds(0, NB)], eew.at[0])
    plsc.subcore_barrier()

    def _scale(rbuf, p, b):
        def grp(j, _):
            ew16 = eew[p, b, pl.ds(j * 16, 16)]
            for r in range(16):
                w = _bcast_lane(ew16, r)
                e = j * 16 + r
                for c8 in range(D // 16):
                    sl = pl.ds(c8 * 16, 16)
                    rbuf[e, sl] = rbuf[e, sl] * w
            return 0
        lax.fori_loop(0, CH // 16, grp, 0)

    for b in range(NB):
        pltpu.async_copy(x_hbm.at[esrc.at[0, b]], rows[b], gsems[b])

    def outer(g, _):
        p = lax.rem(g, 2)
        q = 1 - p

        @pl.when(g < NOUT - 1)
        def _prefetch():
            nxt = (g + 1) * NB
            pltpu.async_copy(src_hbm.at[wid, pl.ds(nxt, NB)], esrc.at[q], esem)
            pltpu.async_copy(dst_hbm.at[wid, pl.ds(nxt, NB)], edst.at[q], esem)
            pltpu.async_copy(ew_hbm.at[wid, pl.ds(nxt, NB)], eew.at[q], esem)

        for b in range(NB):
            pltpu.make_async_copy(
                x_hbm.at[esrc.at[p, b]], rows[b], gsems[b]).wait()
            _scale(rows[b], p, b)
            pltpu.async_copy(rows[b], acc_sh.at[edst.at[p, b]], ssems[b],
                             add=True)

        @pl.when(g < NOUT - 1)
        def _next():
            nxt = (g + 1) * NB
            pltpu.make_async_copy(
                src_hbm.at[wid, pl.ds(nxt, NB)], esrc.at[q], esem).wait()
            pltpu.make_async_copy(
                dst_hbm.at[wid, pl.ds(nxt, NB)], edst.at[q], esem).wait()
            pltpu.make_async_copy(
                ew_hbm.at[wid, pl.ds(nxt, NB)], eew.at[q], esem).wait()
            for b in range(NB):
                pltpu.make_async_copy(
                    rows[b], acc_sh.at[edst.at[p, b]], ssems[b]).wait()
                pltpu.async_copy(x_hbm.at[esrc.at[q, b]], rows[b], gsems[b])
        return 0
    lax.fori_loop(0, NOUT, outer, 0)
    lastp = (NOUT - 1) % 2
    for b in range(NB):
        pltpu.make_async_copy(
            rows[b], acc_sh.at[edst.at[lastp, b]], ssems[b]).wait()

    plsc.subcore_barrier()
    pltpu.sync_copy(acc_sh.at[pl.ds(rbase, RPT), :],
                    out_sum.at[cid, pl.ds(rbase, RPT), :])


_agg = pl.kernel(
    _agg_body,
    out_type=jax.ShapeDtypeStruct((NC, NP, D), jnp.float32),
    mesh=_MESH,
    scratch_types=(
        [pltpu.VMEM((2, NB, CH), jnp.int32),
         pltpu.VMEM((2, NB, CH), jnp.int32),
         pltpu.VMEM((2, NB, CH), jnp.float32),
         pltpu.VMEM_SHARED((NP, D), jnp.float32)]
        + [pltpu.VMEM((CH, D), jnp.float32) for _ in range(NB)]
        + [pltpu.SemaphoreType.DMA for _ in range(2 * NB + 1)]
    ),
    compiler_params=_CP,
)


_RB = 1024


def _sage_dense_body(x_ref, acc_ref, deg_ref, wst_ref, wnt_ref, b_ref, o_ref):
    x = x_ref[...]
    acc = acc_ref[0] + acc_ref[1]
    deg = deg_ref[0, :, 0:1] + deg_ref[1, :, 0:1]
    neigh = acc / jnp.maximum(deg, 1.0)
    h = (jnp.dot(x, wst_ref[...], preferred_element_type=jnp.float32)
         + jnp.dot(neigh, wnt_ref[...], preferred_element_type=jnp.float32)
         + b_ref[...])
    o_ref[...] = jnp.maximum(h, 0.0)


_sage_dense = pl.pallas_call(
    _sage_dense_body,
    grid=(NP // _RB,),
    in_specs=[
        pl.BlockSpec((_RB, D), lambda i: (i, 0)),
        pl.BlockSpec((NC, _RB, D), lambda i: (0, i, 0)),
        pl.BlockSpec((NC, _RB, 16), lambda i: (0, i, 0)),
        pl.BlockSpec((D, D), lambda i: (0, 0)),
        pl.BlockSpec((D, D), lambda i: (0, 0)),
        pl.BlockSpec((1, D), lambda i: (0, 0)),
    ],
    out_specs=pl.BlockSpec((_RB, D), lambda i: (i, 0)),
    out_shape=jax.ShapeDtypeStruct((NP, D), jnp.float32),
)


def _final_body(h_ref, w1t_ref, b1_ref, w2t_ref, b2_ref, o_ref):
    h = jnp.maximum(
        jnp.dot(h_ref[...], w1t_ref[...], preferred_element_type=jnp.float32)
        + b1_ref[...], 0.0)
    o_ref[...] = (jnp.dot(h, w2t_ref[...], preferred_element_type=jnp.float32)
                  + b2_ref[...])


_final = pl.pallas_call(
    _final_body,
    grid=(NP // _RB,),
    in_specs=[
        pl.BlockSpec((_RB, D), lambda i: (i, 0)),
        pl.BlockSpec((D, D), lambda i: (0, 0)),
        pl.BlockSpec((1, D), lambda i: (0, 0)),
        pl.BlockSpec((D, D), lambda i: (0, 0)),
        pl.BlockSpec((1, D), lambda i: (0, 0)),
    ],
    out_specs=pl.BlockSpec((_RB, D), lambda i: (i, 0)),
    out_shape=jax.ShapeDtypeStruct((NP, D), jnp.float32),
)


def kernel(inputs, edge_index, ew, Ws1, Wn1, b1, Ws2, Wn2, b2, Ws3, Wn3, b3,
           lin1_W, lin1_b, lin2_W, lin2_b):
    pad = EP - E
    src = jnp.pad(edge_index[0], (0, pad)).reshape(NW, NCH, CH)
    dst = jnp.pad(edge_index[1], (0, pad),
                  constant_values=N).reshape(NW, NCH, CH)
    ew3 = jnp.pad(ew, (0, pad)).reshape(NW, NCH, CH)
    xp = jnp.pad(inputs, ((0, NP - N), (0, 0)))

    degf = _deg(dst)
    acc1 = _agg(xp, src, dst, ew3)
    h = _sage_dense(xp, acc1, degf, Ws1.T, Wn1.T, b1.reshape(1, D))
    acc2 = _agg(h, src, dst, ew3)
    h = _sage_dense(h, acc2, degf, Ws2.T, Wn2.T, b2.reshape(1, D))
    acc3 = _agg(h, src, dst, ew3)
    h = _sage_dense(h, acc3, degf, Ws3.T, Wn3.T, b3.reshape(1, D))
    h = _final(h, lin1_W.T, lin1_b.reshape(1, D), lin2_W.T, lin2_b.reshape(1, D))
    return h[:N]
